# Initial kernel scaffold; baseline (speedup 1.0000x reference)
import jax
import jax.numpy as jnp
from jax.experimental import pallas as pl


def kernel(tokenized_prompts, token_embedding, ctx):
    raise NotImplementedError("write your pallas kernel here")


# speedup vs baseline: 4.5282x; 4.5282x over previous
import functools

import jax
import jax.numpy as jnp
from jax import lax
from jax.experimental import pallas as pl
from jax.experimental.pallas import tpu as pltpu
from jax.experimental.pallas import tpu_sc as plsc


def kernel(tokenized_prompts, token_embedding, ctx):
    n_cls, ctx_len = tokenized_prompts.shape
    _, d = token_embedding.shape
    n_ctx = ctx.shape[0]

    NC, NS = 2, 16
    NW = NC * NS
    chunk = (-(-n_cls // NW) + 7) // 8 * 8

    mesh = plsc.VectorSubcoreMesh(
        core_axis_name="c", subcore_axis_name="s", num_cores=NC, num_subcores=NS
    )

    @functools.partial(
        pl.kernel,
        out_type=jax.ShapeDtypeStruct((n_cls, ctx_len, d), jnp.float32),
        mesh=mesh,
        scratch_types=[
            pltpu.VMEM((chunk, ctx_len), jnp.int32),
            pltpu.VMEM((ctx_len, d), jnp.float32),
            pltpu.VMEM((n_ctx, d), jnp.float32),
            pltpu.VMEM((ctx_len,), jnp.int32),
            pltpu.SemaphoreType.DMA,
            pltpu.SemaphoreType.DMA,
        ],
    )
    def _k(tp_hbm, table_hbm, ctx_hbm, out_hbm, idx_v, rows_v, ctx_v, rid_v, sem, sem2):
        w = lax.axis_index("s") * NC + lax.axis_index("c")
        start = w * chunk
        cnt = jnp.clip(n_cls - start, 0, chunk)
        stage = jnp.minimum(start, n_cls - chunk)
        off = start - stage
        pltpu.sync_copy(ctx_hbm, ctx_v)
        pltpu.sync_copy(tp_hbm.at[pl.ds(stage, chunk)], idx_v)
        iota16 = lax.iota(jnp.int32, 16)
        for k0 in range(0, ctx_len - 16, 16):
            rid_v[pl.ds(k0, 16)] = k0 + iota16
        rid_v[pl.ds(ctx_len - 16, 16)] = (ctx_len - 16) + iota16

        def body(j, carry):
            pltpu.async_copy(table_hbm.at[idx_v.at[off + j]], rows_v, sem).wait()
            for r in range(n_ctx):
                for k in range(d // 16):
                    rows_v[1 + r, pl.ds(16 * k, 16)] = ctx_v[r, pl.ds(16 * k, 16)]
            pltpu.async_copy(rows_v, out_hbm.at[start + j].at[rid_v], sem2).wait()
            return carry

        lax.fori_loop(0, cnt, body, 0)

    return _k(tokenized_prompts, token_embedding, ctx)


# speedup vs baseline: 4.9682x; 1.0972x over previous
import functools

import jax
import jax.numpy as jnp
from jax import lax
from jax.experimental import pallas as pl
from jax.experimental.pallas import tpu as pltpu
from jax.experimental.pallas import tpu_sc as plsc


def kernel(tokenized_prompts, token_embedding, ctx):
    n_cls, ctx_len = tokenized_prompts.shape
    _, d = token_embedding.shape
    n_ctx = ctx.shape[0]

    NC, NS = 2, 16
    NW = NC * NS
    chunk = (-(-n_cls // NW) + 7) // 8 * 8

    mesh = plsc.VectorSubcoreMesh(
        core_axis_name="c", subcore_axis_name="s", num_cores=NC, num_subcores=NS
    )

    @functools.partial(
        pl.kernel,
        out_type=jax.ShapeDtypeStruct((n_cls, ctx_len, d), jnp.float32),
        mesh=mesh,
        scratch_types=[
            pltpu.VMEM((chunk, ctx_len), jnp.int32),
            pltpu.VMEM((2, ctx_len, d), jnp.float32),
            pltpu.VMEM((n_ctx, d), jnp.float32),
            pltpu.VMEM((ctx_len,), jnp.int32),
            pltpu.SemaphoreType.DMA,
            pltpu.SemaphoreType.DMA,
        ],
    )
    def _k(tp_hbm, table_hbm, ctx_hbm, out_hbm, idx_v, rows_v, ctx_v, rid_v, sem, sem2):
        w = lax.axis_index("s") * NC + lax.axis_index("c")
        start = w * chunk
        cnt = jnp.clip(n_cls - start, 0, chunk)
        stage = jnp.minimum(start, n_cls - chunk)
        off = start - stage
        pltpu.sync_copy(ctx_hbm, ctx_v)
        pltpu.sync_copy(tp_hbm.at[pl.ds(stage, chunk)], idx_v)
        iota16 = lax.iota(jnp.int32, 16)
        for k0 in range(0, ctx_len - 16, 16):
            rid_v[pl.ds(k0, 16)] = k0 + iota16
        rid_v[pl.ds(ctx_len - 16, 16)] = (ctx_len - 16) + iota16

        def gather(j, b):
            return pltpu.make_async_copy(
                table_hbm.at[idx_v.at[off + j]], rows_v.at[b], sem
            )

        def scatter(j, b):
            return pltpu.make_async_copy(
                rows_v.at[b], out_hbm.at[start + j].at[rid_v], sem2
            )

        @pl.when(cnt > 0)
        def _run():
            gather(0, 0).start()

            def body(j, carry):
                b = lax.rem(j, 2)

                @pl.when(j > 0)
                def _():
                    scatter(j - 1, 1 - b).wait()

                gather(j, b).wait()

                @pl.when(j + 1 < cnt)
                def _():
                    gather(j + 1, 1 - b).start()

                for r in range(n_ctx):
                    for k in range(d // 16):
                        rows_v[b, 1 + r, pl.ds(16 * k, 16)] = ctx_v[r, pl.ds(16 * k, 16)]
                scatter(j, b).start()
                return carry

            lax.fori_loop(0, cnt, body, 0)
            scatter(cnt - 1, lax.rem(cnt - 1, 2)).wait()

    return _k(tokenized_prompts, token_embedding, ctx)
